# baseline (speedup 1.0000x reference)
import jax
import jax.numpy as jnp
from jax import lax
from jax.experimental import pallas as pl
from jax.experimental.pallas import tpu as pltpu
from jax.experimental.pallas import tpu_sc as plsc

_N = 50000
_E = 800000
_HID = 128

_NC = 2
_NS = 16
_NT = _NC * _NS

_NPAD = 50176
_ROWS = _NPAD // 128
_SLICE = _NPAD // _NS
_EPT = _E // _NT
_EVEC = _EPT // 16
_ESTREAM = 25024

_RPP = 3584
_DUMP = _RPP
_EPS = _E // _NS
_SCH = 2000
_NCH = _EPS // _SCH
_CAP = 8192
_BATCH = 512
_WB = _RPP // _NS


def _segsum_body(table_hbm, edge_hbm, out_hbm,
                 table_v, src_v, dst_v, vals_v, acc_sh):
    cid = lax.axis_index("c")
    sid = lax.axis_index("s")
    wid = cid * _NS + sid

    pltpu.sync_copy(table_hbm, table_v)

    def _zero_body(i, carry):
        vals_v[pl.ds(i * 16, 16)] = jnp.zeros((16,), jnp.float32)
        return carry

    lax.fori_loop(0, _SLICE // 16, _zero_body, 0)
    pltpu.sync_copy(vals_v.at[pl.ds(0, _SLICE)],
                    acc_sh.at[pl.ds(sid * _SLICE, _SLICE)])
    plsc.subcore_barrier()

    base = wid * _EPT
    pltpu.sync_copy(edge_hbm.at[pl.ds(base, _EPT)], src_v.at[pl.ds(0, _EPT)])
    pltpu.sync_copy(edge_hbm.at[pl.ds(_E + base, _EPT)],
                    dst_v.at[pl.ds(0, _EPT)])

    def _gather_body(j, carry):
        o = pl.multiple_of(j * 16, 16)
        idx = jnp.clip(src_v[pl.ds(o, 16)], 0, _N - 1)
        vals_v[pl.ds(o, 16)] = plsc.load_gather(table_v, [idx])
        return carry

    lax.fori_loop(0, _EVEC + 1, _gather_body, 0)

    keep = lax.iota(jnp.int32, 16) < 8
    o = _EVEC * 16
    dst_v[pl.ds(o, 16)] = jnp.where(keep, dst_v[pl.ds(o, 16)], 0)
    vals_v[pl.ds(o, 16)] = jnp.where(keep, vals_v[pl.ds(o, 16)],
                                     jnp.zeros((16,), jnp.float32))
    dst_v[pl.ds(o + 16, 16)] = jnp.zeros((16,), jnp.int32)
    vals_v[pl.ds(o + 16, 16)] = jnp.zeros((16,), jnp.float32)

    pltpu.sync_copy(vals_v.at[pl.ds(0, _ESTREAM)], acc_sh.at[dst_v], add=True)

    plsc.subcore_barrier()
    pltpu.sync_copy(acc_sh.at[pl.ds(sid * _SLICE, _SLICE)],
                    vals_v.at[pl.ds(0, _SLICE)])
    pltpu.sync_copy(vals_v.at[pl.ds(0, _SLICE)],
                    out_hbm.at[pl.ds(cid * _NPAD + sid * _SLICE, _SLICE)])


_segsum = pl.kernel(
    _segsum_body,
    out_type=jax.ShapeDtypeStruct((_NC * _NPAD,), jnp.float32),
    mesh=plsc.VectorSubcoreMesh(core_axis_name="c", subcore_axis_name="s"),
    compiler_params=pltpu.CompilerParams(needs_layout_passes=False),
    scratch_types=[
        pltpu.VMEM((_NPAD,), jnp.float32),
        pltpu.VMEM((_EPT + 16,), jnp.int32),
        pltpu.VMEM((_ESTREAM,), jnp.int32),
        pltpu.VMEM((_ESTREAM,), jnp.float32),
        pltpu.VMEM_SHARED((_NPAD,), jnp.float32),
    ],
)


def _vecsum_body(h_hbm, edge_hbm, out_hbm,
                 srcc, dstc, scmp, dcmp, bsrc, bdst, rows_v, zbuf, acc_sh):
    cid = lax.axis_index("c")
    sid = lax.axis_index("s")

    def _zb(i, carry):
        zbuf[i // 8, pl.ds((i % 8) * 16, 16)] = jnp.zeros((16,), jnp.float32)
        return carry

    lax.fori_loop(0, 64 * 8, _zb, 0)

    for p in range(7):
        lo = (2 * p + cid) * _RPP

        for q in range(3):
            pltpu.sync_copy(zbuf, acc_sh.at[pl.ds(sid * _WB + q * 64, 64)])
        pltpu.sync_copy(zbuf.at[pl.ds(0, 32)],
                        acc_sh.at[pl.ds(sid * _WB + 192, 32)])
        plsc.subcore_barrier()

        def _pf(i, carry):
            scmp[pl.ds(i * 16, 16)] = jnp.zeros((16,), jnp.int32)
            dcmp[pl.ds(i * 16, 16)] = jnp.full((16,), _DUMP, jnp.int32)
            return carry

        lax.fori_loop(0, _CAP // 16, _pf, 0)

        def _chunk(c, off):
            e0 = sid * _EPS + c * _SCH
            pltpu.sync_copy(edge_hbm.at[pl.ds(e0, _SCH)], srcc)
            pltpu.sync_copy(edge_hbm.at[pl.ds(_E + e0, _SCH)], dstc)

            def _vec(j, o):
                d = dstc[pl.ds(j * 16, 16)]
                sv = srcc[pl.ds(j * 16, 16)]
                m = (d >= lo) & (d < lo + _RPP)
                plsc.store_compressed(dcmp.at[pl.ds(o, 16)], d - lo, mask=m)
                plsc.store_compressed(scmp.at[pl.ds(o, 16)], sv, mask=m)
                cnt = jnp.max(plsc.all_reduce_population_count(m))
                return jnp.minimum(o + cnt, _CAP - 16)

            return lax.fori_loop(0, _SCH // 16, _vec, off)

        off = lax.fori_loop(0, _NCH, _chunk, jnp.int32(0))

        def _batch(b, carry):
            @pl.when(b * _BATCH < off)
            def _():
                def _cp(k, c2):
                    bsrc[pl.ds(k * 16, 16)] = scmp[pl.ds(b * _BATCH + k * 16, 16)]
                    bdst[pl.ds(k * 16, 16)] = dcmp[pl.ds(b * _BATCH + k * 16, 16)]
                    return c2

                lax.fori_loop(0, _BATCH // 16, _cp, 0)
                pltpu.sync_copy(h_hbm.at[bsrc], rows_v)
                pltpu.sync_copy(rows_v, acc_sh.at[bdst], add=True)

            return carry

        lax.fori_loop(0, _CAP // _BATCH, _batch, 0)
        plsc.subcore_barrier()

        row0 = (2 * p + cid) * _RPP + sid * _WB
        pltpu.sync_copy(acc_sh.at[pl.ds(sid * _WB, _WB)],
                        rows_v.at[pl.ds(0, _WB)])
        pltpu.sync_copy(rows_v.at[pl.ds(0, _WB)],
                        out_hbm.at[pl.ds(row0, _WB)])
        plsc.subcore_barrier()


_vecsum = pl.kernel(
    _vecsum_body,
    out_type=jax.ShapeDtypeStruct((_NPAD, _HID), jnp.float32),
    mesh=plsc.VectorSubcoreMesh(core_axis_name="c", subcore_axis_name="s"),
    compiler_params=pltpu.CompilerParams(needs_layout_passes=False),
    scratch_types=[
        pltpu.VMEM((_SCH,), jnp.int32),
        pltpu.VMEM((_SCH,), jnp.int32),
        pltpu.VMEM((_CAP,), jnp.int32),
        pltpu.VMEM((_CAP,), jnp.int32),
        pltpu.VMEM((_BATCH,), jnp.int32),
        pltpu.VMEM((_BATCH,), jnp.int32),
        pltpu.VMEM((_BATCH, _HID), jnp.float32),
        pltpu.VMEM((64, _HID), jnp.float32),
        pltpu.VMEM_SHARED((_RPP + 1, _HID), jnp.float32),
    ],
)


_RB = 8
_GRID = _ROWS // _RB


def _rb16(v):
    return v.astype(jnp.bfloat16).astype(jnp.float32)


def _dense_kern(spart_ref, x_ref, wa_ref, wb_ref, wc_ref, wr2_ref,
                h_ref, u_ref):
    s = (spart_ref[0] + spart_ref[1])[:, :, None]
    xb = x_ref[...][:, :, None]
    h = jnp.maximum(s * wa_ref[...] + wb_ref[...] + xb * wc_ref[...], 0.0)
    h_ref[...] = h
    u_ref[...] = jnp.sum(_rb16(h) * _rb16(wr2_ref[...]), axis=2)


def _dense(s_part, x2d, wa, wb, wc, wr2):
    wspec = pl.BlockSpec((1, 1, _HID), lambda i: (0, 0, 0))
    rspec = pl.BlockSpec((_RB, 128), lambda i: (i, 0))
    return pl.pallas_call(
        _dense_kern,
        grid=(_GRID,),
        in_specs=[pl.BlockSpec((_NC, _RB, 128), lambda i: (0, i, 0)),
                  rspec, wspec, wspec, wspec, wspec],
        out_specs=[pl.BlockSpec((_RB, 128, _HID), lambda i: (i, 0, 0)),
                   rspec],
        out_shape=[jax.ShapeDtypeStruct((_ROWS, 128, _HID), jnp.float32),
                   jax.ShapeDtypeStruct((_ROWS, 128), jnp.float32)],
    )(s_part, x2d, wa, wb, wc, wr2)


def _final_kern(agg_ref, u_ref, w2_ref, b2_ref, emb_ref, upd_ref):
    e = (jnp.sum(_rb16(agg_ref[...]) * _rb16(w2_ref[...]), axis=2)
         + b2_ref[0, 0] + u_ref[...])
    emb_ref[...] = e
    upd_ref[...] = jnp.maximum(e, 0.0)


def _final(agg3, u2d, w2, b2):
    rspec = pl.BlockSpec((_RB, 128), lambda i: (i, 0))
    return pl.pallas_call(
        _final_kern,
        grid=(_GRID,),
        in_specs=[pl.BlockSpec((_RB, 128, _HID), lambda i: (i, 0, 0)),
                  rspec,
                  pl.BlockSpec((1, 1, _HID), lambda i: (0, 0, 0)),
                  pl.BlockSpec((1, 1), lambda i: (0, 0))],
        out_specs=[rspec, rspec],
        out_shape=[jax.ShapeDtypeStruct((_ROWS, 128), jnp.float32)] * 2,
    )(agg3, u2d, w2, b2)


def kernel(x, edge_index, W_rel1, b_rel1, W_root1, W_rel2, b_rel2, W_root2):
    x_pad = jnp.pad(x[:, 0], (0, _NPAD - _N))
    edge_flat = edge_index.reshape(2 * _E)

    s_part = _segsum(x_pad, edge_flat)

    wa = W_rel1.reshape(1, 1, _HID)
    wb = b_rel1.reshape(1, 1, _HID)
    wc = W_root1.reshape(1, 1, _HID)
    wr2 = W_root2.reshape(1, 1, _HID)
    h3, u2d = _dense(s_part.reshape(_NC, _ROWS, 128),
                     x_pad.reshape(_ROWS, 128), wa, wb, wc, wr2)

    agg2 = _vecsum(h3.reshape(_NPAD, _HID), edge_flat)

    emb2d, upd2d = _final(agg2.reshape(_ROWS, 128, _HID), u2d,
                          W_rel2.reshape(1, 1, _HID), b_rel2.reshape(1, 1))
    emb = emb2d.reshape(_NPAD, 1)[:_N]
    upd = upd2d.reshape(_NPAD, 1)[:_N]
    return (emb, upd)
